# baseline (speedup 1.0000x reference)
import functools

import jax
import jax.numpy as jnp
import numpy as np
from jax import lax
from jax.experimental import pallas as pl
from jax.experimental.pallas import tpu as pltpu
from jax.experimental.pallas import tpu_sc as plsc

N_NODES = 10000
N_EDGES = 320000
D = 128

NC = 2
NS = 16
NW = NC * NS
EPW = N_EDGES // NW
CH = 80
NCHUNK = EPW // CH
NPAD = 10240
RPT = NPAD // NS
DEGW = 16


def _fill2d(ref, nrows, ncols, val):
    def row(i, c):
        def col(k, c2):
            ref[i, pl.ds(k * 16, 16)] = jnp.full((16,), val, jnp.float32)
            return c2
        return lax.fori_loop(0, ncols // 16, col, c)
    lax.fori_loop(0, nrows, row, 0)


NBUF = 4


def _sc_deg(dstr):
    mesh = plsc.VectorSubcoreMesh(core_axis_name="c", subcore_axis_name="s")

    @functools.partial(
        pl.kernel,
        out_type=jax.ShapeDtypeStruct((NW, NPAD), jnp.float32),
        mesh=mesh,
        scratch_types=[
            pltpu.VMEM((EPW,), jnp.int32),
            pltpu.VMEM((NPAD,), jnp.float32),
        ],
        compiler_params=pltpu.CompilerParams(needs_layout_passes=False),
    )
    def k(dst_hbm, deg_hbm, didx, dega):
        c = lax.axis_index("c")
        s = lax.axis_index("s")
        wid = s * NC + c

        def z(j, carry):
            dega[pl.ds(j * 16, 16)] = jnp.zeros((16,), jnp.float32)
            return carry
        lax.fori_loop(0, NPAD // 16, z, 0)

        pltpu.sync_copy(dst_hbm.at[wid], didx)

        ones = jnp.ones((16,), jnp.float32)

        def sub(k2, c2):
            idx = didx[pl.ds(k2 * 16, 16)]
            plsc.addupdate_scatter(dega, [idx], ones)
            return c2
        lax.fori_loop(0, EPW // 16, sub, 0)

        pltpu.sync_copy(dega, deg_hbm.at[wid])

    return k(dstr)


def _sc_agg(hpk, src, dst):
    mesh = plsc.VectorSubcoreMesh(core_axis_name="c", subcore_axis_name="s")
    NF = 2

    @functools.partial(
        pl.kernel,
        out_type=jax.ShapeDtypeStruct((NC, NPAD, D), jnp.float32),
        mesh=mesh,
        scratch_types=[
            pltpu.VMEM((NBUF, CH), jnp.int32),
            pltpu.VMEM((NBUF, CH), jnp.int32),
            pltpu.VMEM((NBUF, CH, D // 2), jnp.int32),
            pltpu.VMEM((NF, CH, D), jnp.float32),
            pltpu.VMEM((8, D), jnp.float32),
            pltpu.VMEM_SHARED((NPAD, D), jnp.float32),
            pltpu.SemaphoreType.DMA,
            pltpu.SemaphoreType.DMA,
            pltpu.SemaphoreType.DMA,
            pltpu.SemaphoreType.DMA,
        ],
        compiler_params=pltpu.CompilerParams(needs_layout_passes=False,
                                             use_tc_tiling_on_sc=False),
    )
    def body(h_hbm, src_hbm, dst_hbm, out_hbm,
             sidx, didx, rbf, rows, zmain, acc, s_isem, d_isem, gsem, ssem):
        c = lax.axis_index("c")
        s = lax.axis_index("s")
        wid = s * NC + c

        _fill2d(zmain, 8, D, 0.0)

        def z1(j, carry):
            pltpu.sync_copy(zmain, acc.at[pl.ds(s * RPT + j * 8, 8)])
            return carry
        lax.fori_loop(0, RPT // 8, z1, 0)

        plsc.subcore_barrier()

        ebase = wid * EPW

        def is_start(jj, b):
            pltpu.async_copy(src_hbm.at[pl.ds(ebase + jj * CH, CH)],
                             sidx.at[b], s_isem)

        def is_drain(b):
            pltpu.make_async_copy(src_hbm.at[pl.ds(0, CH)],
                                  sidx.at[b], s_isem).wait()

        def id_start(jj, b):
            pltpu.async_copy(dst_hbm.at[pl.ds(ebase + jj * CH, CH)],
                             didx.at[b], d_isem)

        def id_drain(b):
            pltpu.make_async_copy(dst_hbm.at[pl.ds(0, CH)],
                                  didx.at[b], d_isem).wait()

        def g_start(b):
            pltpu.async_copy(h_hbm.at[sidx.at[b]], rbf.at[b], gsem)

        def g_wait(b):
            pltpu.make_async_copy(h_hbm.at[sidx.at[b]], rbf.at[b],
                                  gsem).wait()

        def widen(b, f):
            mask = jnp.full((16,), -65536, jnp.int32)

            def row(r, carry):
                for g in range(D // 32):
                    v = rbf[b, r, pl.ds(g * 16, 16)]
                    lo = plsc.bitcast(jnp.left_shift(v, 16), jnp.float32)
                    hi = plsc.bitcast(jnp.bitwise_and(v, mask), jnp.float32)
                    rows[f, r, pl.ds(g * 16, 16)] = lo
                    rows[f, r, pl.ds(D // 2 + g * 16, 16)] = hi
                return carry
            lax.fori_loop(0, CH, row, 0)

        def s_start(b, f):
            pltpu.async_copy(rows.at[f], acc.at[didx.at[b]], ssem, add=True)

        def s_drain(b, f):
            pltpu.make_async_copy(rows.at[f], acc.at[didx.at[b]],
                                  ssem).wait()

        for j in range(NBUF):
            is_start(jnp.int32(j), j)
            id_start(jnp.int32(j), j)
        for j in range(NBUF - 1):
            is_drain(j)
            g_start(j)
        g_wait(0)
        widen(0, 0)
        id_drain(0)
        s_start(0, 0)
        is_drain(3)
        g_start(3)
        is_start(jnp.int32(NBUF), 0)

        def quad(t, carry):
            for b4 in range(NBUF):
                jj = 1 + t * NBUF + b4
                b = (1 + b4) % NBUF
                bp = (b - 1) % NBUF
                f = (1 + b4) % NF
                fp = 1 - f
                g_wait(b)
                widen(b, f)
                id_drain(b)
                s_drain(bp, fp)
                s_start(b, f)

                @pl.when(jj + 3 < NCHUNK)
                def _():
                    id_start(jj + 3, bp)
                    is_drain(bp)
                    g_start(bp)

                @pl.when(jj + 4 < NCHUNK)
                def _():
                    is_start(jj + 4, b)
            return carry
        lax.fori_loop(0, (NCHUNK - 1) // NBUF, quad, 0)

        s_drain(0, 0)

        plsc.subcore_barrier()

        pltpu.sync_copy(acc.at[pl.ds(s * RPT, RPT)],
                        out_hbm.at[c].at[pl.ds(s * RPT, RPT)])

    return body(hpk, src, dst)


def _tc_dense(h, p, degs, w_self, w_neigh_perm, b, relu, emit_bf16):
    R = 1024

    def body(h_ref, p0_ref, p1_ref, d_ref, ws_ref, wn_ref, b_ref, *outs):
        deg = jnp.sum(d_ref[...], axis=0)
        recip = 1.0 / jnp.maximum(deg, 1.0)
        neigh = (p0_ref[0] + p1_ref[0]) * recip[:, None]
        acc = jnp.dot(h_ref[...], ws_ref[...],
                      preferred_element_type=jnp.float32)
        acc = acc + jnp.dot(neigh, wn_ref[...],
                            preferred_element_type=jnp.float32)
        acc = acc + b_ref[...]
        if relu:
            acc = jnp.maximum(acc, 0.0)
        outs[0][...] = acc
        if emit_bf16:
            outs[1][...] = acc.astype(jnp.bfloat16)

    out_shape = [jax.ShapeDtypeStruct((NPAD, D), jnp.float32)]
    out_specs = [pl.BlockSpec((R, D), lambda i: (i, 0))]
    if emit_bf16:
        out_shape.append(jax.ShapeDtypeStruct((NPAD, D), jnp.bfloat16))
        out_specs.append(pl.BlockSpec((R, D), lambda i: (i, 0)))

    res = pl.pallas_call(
        body,
        grid=(NPAD // R,),
        in_specs=[
            pl.BlockSpec((R, D), lambda i: (i, 0)),
            pl.BlockSpec((1, R, D), lambda i: (0, i, 0)),
            pl.BlockSpec((1, R, D), lambda i: (1, i, 0)),
            pl.BlockSpec((NW, R), lambda i: (0, i)),
            pl.BlockSpec((D, D), lambda i: (0, 0)),
            pl.BlockSpec((D, D), lambda i: (0, 0)),
            pl.BlockSpec((1, D), lambda i: (0, 0)),
        ],
        out_specs=out_specs,
        out_shape=out_shape,
    )(h, p, p, degs, w_self, w_neigh_perm, b.reshape(1, D))
    return res if emit_bf16 else res[0]


def _pack_bf16(a_bf16):
    return jax.lax.bitcast_convert_type(
        a_bf16.reshape(NPAD, D // 2, 2), jnp.int32)


_PERM = np.array(
    [32 * (j // 16) + 2 * (j % 16) for j in range(D // 2)]
    + [32 * (j // 16) + 2 * (j % 16) + 1 for j in range(D // 2)],
    dtype=np.int32)


def kernel(x, edge_index, W1_self, W1_neigh, b1, W2_self, W2_neigh, b2):
    src = edge_index[0]
    dst = edge_index[1]

    dstr = dst.reshape(NW, EPW)
    x_pad = jnp.concatenate(
        [x, jnp.zeros((NPAD - N_NODES, D), jnp.float32)], axis=0)
    xpk = _pack_bf16(x_pad.astype(jnp.bfloat16))
    w1n = W1_neigh[_PERM, :]
    w2n = W2_neigh[_PERM, :]

    degp = _sc_deg(dstr)
    p1 = _sc_agg(xpk, src, dst)
    h1, h1b = _tc_dense(x_pad, p1, degp, W1_self, w1n, b1,
                        relu=True, emit_bf16=True)
    p2 = _sc_agg(_pack_bf16(h1b), src, dst)
    h2 = _tc_dense(h1, p2, degp, W2_self, w2n, b2,
                   relu=False, emit_bf16=False)
    return h2[:N_NODES]


# speedup vs baseline: 2.0630x; 2.0630x over previous
import functools

import jax
import jax.numpy as jnp
import numpy as np
from jax import lax
from jax.experimental import pallas as pl
from jax.experimental.pallas import tpu as pltpu
from jax.experimental.pallas import tpu_sc as plsc

N_NODES = 10000
N_EDGES = 320000
D = 128

NC = 2
NS = 16
NW = NC * NS
EPW = N_EDGES // NW
CH = 80
NCHUNK = EPW // CH
NPAD = 10240
RPT = NPAD // NS
DEGW = 16


def _fill2d(ref, nrows, ncols, val):
    def row(i, c):
        def col(k, c2):
            ref[i, pl.ds(k * 16, 16)] = jnp.full((16,), val, jnp.float32)
            return c2
        return lax.fori_loop(0, ncols // 16, col, c)
    lax.fori_loop(0, nrows, row, 0)


NBUF = 4


def _sc_deg(dstr):
    mesh = plsc.VectorSubcoreMesh(core_axis_name="c", subcore_axis_name="s")

    @functools.partial(
        pl.kernel,
        out_type=jax.ShapeDtypeStruct((NW, NPAD), jnp.float32),
        mesh=mesh,
        scratch_types=[
            pltpu.VMEM((EPW,), jnp.int32),
            pltpu.VMEM((NPAD,), jnp.float32),
        ],
        compiler_params=pltpu.CompilerParams(needs_layout_passes=False),
    )
    def k(dst_hbm, deg_hbm, didx, dega):
        c = lax.axis_index("c")
        s = lax.axis_index("s")
        wid = s * NC + c

        def z(j, carry):
            dega[pl.ds(j * 16, 16)] = jnp.zeros((16,), jnp.float32)
            return carry
        lax.fori_loop(0, NPAD // 16, z, 0)

        pltpu.sync_copy(dst_hbm.at[wid], didx)

        ones = jnp.ones((16,), jnp.float32)

        def sub(k2, c2):
            idx = didx[pl.ds(k2 * 16, 16)]
            plsc.addupdate_scatter(dega, [idx], ones)
            return c2
        lax.fori_loop(0, EPW // 16, sub, 0)

        pltpu.sync_copy(dega, deg_hbm.at[wid])

    return k(dstr)


def _sc_agg(h, src, dst):
    mesh = plsc.VectorSubcoreMesh(core_axis_name="c", subcore_axis_name="s")
    CH2 = CH // 2

    @functools.partial(
        pl.kernel,
        out_type=jax.ShapeDtypeStruct((NC, NPAD, D), jnp.float32),
        mesh=mesh,
        scratch_types=[
            pltpu.VMEM((NBUF, CH), jnp.int32),
            pltpu.VMEM((NBUF, CH), jnp.int32),
            pltpu.VMEM((NBUF, CH, D), jnp.float32),
            pltpu.VMEM((8, D), jnp.float32),
            pltpu.VMEM_SHARED((NPAD, D), jnp.float32),
            pltpu.SemaphoreType.DMA,
            pltpu.SemaphoreType.DMA,
            pltpu.SemaphoreType.DMA,
            pltpu.SemaphoreType.DMA,
        ],
    )
    def body(h_hbm, src_hbm, dst_hbm, out_hbm,
             sidx, didx, rows, zmain, acc, s_isem, d_isem, gsem, ssem):
        c = lax.axis_index("c")
        s = lax.axis_index("s")
        wid = s * NC + c

        _fill2d(zmain, 8, D, 0.0)

        def z1(j, carry):
            pltpu.sync_copy(zmain, acc.at[pl.ds(s * RPT + j * 8, 8)])
            return carry
        lax.fori_loop(0, RPT // 8, z1, 0)

        plsc.subcore_barrier()

        ebase = wid * EPW

        def is_start(jj, b):
            pltpu.async_copy(src_hbm.at[pl.ds(ebase + jj * CH, CH)],
                             sidx.at[b], s_isem)

        def is_drain(b):
            pltpu.make_async_copy(src_hbm.at[pl.ds(0, CH)],
                                  sidx.at[b], s_isem).wait()

        def id_start(jj, b):
            pltpu.async_copy(dst_hbm.at[pl.ds(ebase + jj * CH, CH)],
                             didx.at[b], d_isem)

        def id_drain(b):
            pltpu.make_async_copy(dst_hbm.at[pl.ds(0, CH)],
                                  didx.at[b], d_isem).wait()

        def g_start(b):
            for q in range(2):
                pltpu.async_copy(
                    h_hbm.at[sidx.at[b].at[pl.ds(q * CH2, CH2)]],
                    rows.at[b].at[pl.ds(q * CH2, CH2)], gsem)

        def g_wait(b):
            for q in range(2):
                pltpu.make_async_copy(
                    h_hbm.at[sidx.at[b].at[pl.ds(0, CH2)]],
                    rows.at[b].at[pl.ds(q * CH2, CH2)], gsem).wait()

        def s_start(b):
            pltpu.async_copy(rows.at[b], acc.at[didx.at[b]], ssem, add=True)

        def s_drain(b):
            pltpu.make_async_copy(rows.at[b], acc.at[didx.at[b]],
                                  ssem).wait()

        for j in range(NBUF):
            is_start(jnp.int32(j), j)
            id_start(jnp.int32(j), j)
        for j in range(NBUF - 1):
            is_drain(j)
            g_start(j)
        g_wait(0)
        id_drain(0)
        s_start(0)
        is_drain(3)
        g_start(3)
        is_start(jnp.int32(NBUF), 0)

        def quad(t, carry):
            for b4 in range(NBUF):
                jj = 1 + t * NBUF + b4
                b = (1 + b4) % NBUF
                bp = (b - 1) % NBUF
                g_wait(b)
                id_drain(b)
                s_start(b)
                s_drain(bp)

                @pl.when(jj + 3 < NCHUNK)
                def _():
                    id_start(jj + 3, bp)
                    is_drain(bp)
                    g_start(bp)

                @pl.when(jj + 4 < NCHUNK)
                def _():
                    is_start(jj + 4, b)
            return carry
        lax.fori_loop(0, (NCHUNK - 1) // NBUF, quad, 0)

        s_drain(0)

        plsc.subcore_barrier()

        pltpu.sync_copy(acc.at[pl.ds(s * RPT, RPT)],
                        out_hbm.at[c].at[pl.ds(s * RPT, RPT)])

    return body(h, src, dst)


def _tc_dense(h, p, degs, w_self, w_neigh_perm, b, relu, emit_bf16):
    R = 1024

    def body(h_ref, p0_ref, p1_ref, d_ref, ws_ref, wn_ref, b_ref, *outs):
        deg = jnp.sum(d_ref[...], axis=0)
        recip = 1.0 / jnp.maximum(deg, 1.0)
        neigh = (p0_ref[0] + p1_ref[0]) * recip[:, None]
        acc = jnp.dot(h_ref[...], ws_ref[...],
                      preferred_element_type=jnp.float32)
        acc = acc + jnp.dot(neigh, wn_ref[...],
                            preferred_element_type=jnp.float32)
        acc = acc + b_ref[...]
        if relu:
            acc = jnp.maximum(acc, 0.0)
        outs[0][...] = acc
        if emit_bf16:
            outs[1][...] = acc.astype(jnp.bfloat16)

    out_shape = [jax.ShapeDtypeStruct((NPAD, D), jnp.float32)]
    out_specs = [pl.BlockSpec((R, D), lambda i: (i, 0))]
    if emit_bf16:
        out_shape.append(jax.ShapeDtypeStruct((NPAD, D), jnp.bfloat16))
        out_specs.append(pl.BlockSpec((R, D), lambda i: (i, 0)))

    res = pl.pallas_call(
        body,
        grid=(NPAD // R,),
        in_specs=[
            pl.BlockSpec((R, D), lambda i: (i, 0)),
            pl.BlockSpec((1, R, D), lambda i: (0, i, 0)),
            pl.BlockSpec((1, R, D), lambda i: (1, i, 0)),
            pl.BlockSpec((NW, R), lambda i: (0, i)),
            pl.BlockSpec((D, D), lambda i: (0, 0)),
            pl.BlockSpec((D, D), lambda i: (0, 0)),
            pl.BlockSpec((1, D), lambda i: (0, 0)),
        ],
        out_specs=out_specs,
        out_shape=out_shape,
    )(h, p, p, degs, w_self, w_neigh_perm, b.reshape(1, D))
    return res if emit_bf16 else res[0]


def _pack_bf16(a_bf16):
    return jax.lax.bitcast_convert_type(
        a_bf16.reshape(NPAD, D // 2, 2), jnp.int32)


_PERM = np.array(
    [32 * (j // 16) + 2 * (j % 16) for j in range(D // 2)]
    + [32 * (j // 16) + 2 * (j % 16) + 1 for j in range(D // 2)],
    dtype=np.int32)


def kernel(x, edge_index, W1_self, W1_neigh, b1, W2_self, W2_neigh, b2):
    src = edge_index[0]
    dst = edge_index[1]

    dstr = dst.reshape(NW, EPW)
    x_pad = jnp.concatenate(
        [x, jnp.zeros((NPAD - N_NODES, D), jnp.float32)], axis=0)
    degp = _sc_deg(dstr)
    p1 = _sc_agg(x_pad, src, dst)
    h1 = _tc_dense(x_pad, p1, degp, W1_self, W1_neigh, b1,
                   relu=True, emit_bf16=False)
    p2 = _sc_agg(h1, src, dst)
    h2 = _tc_dense(h1, p2, degp, W2_self, W2_neigh, b2,
                   relu=False, emit_bf16=False)
    return h2[:N_NODES]
